# Initial kernel scaffold; baseline (speedup 1.0000x reference)
import jax
import jax.numpy as jnp
from jax.experimental import pallas as pl


def kernel(x_nchw, w_c1, b_c1, w_c2, b_c2, w_c3, b_c3, w_fc1, b_fc1, w_fc2, b_fc2, w_fc3, b_fc3):
    raise NotImplementedError("write your pallas kernel here")


# speedup vs baseline: 44.5600x; 44.5600x over previous
import numpy as np
import jax
import jax.numpy as jnp
from jax.experimental import pallas as pl
from jax.experimental.pallas import tpu as pltpu

_F32 = jnp.float32


def _shift_matrix(win, pout, ncol_pad):
    e = np.zeros((3, win, 2, pout), np.float32)
    for dx in range(3):
        for q in range(2):
            for p in range(pout):
                e[dx, 2 * p + q + dx, q, p] = 1.0
    return e


def _conv_toeplitz(w_mat, cin, win, pout, cin_major, pad_lanes):
    cout = w_mat.shape[1]
    taps = w_mat.reshape(3, 3, cin, cout)
    e = jnp.asarray(_shift_matrix(win, pout, None))
    blocks = []
    for dy in range(3):
        t = taps[dy]
        if cin_major:
            wd = jnp.einsum("dxqp,dco->cxqpo", e, t)
            wd = wd.reshape(cin * win, 2, pout * cout)
        else:
            wd = jnp.einsum("dxqp,dco->xcqpo", e, t)
            wd = wd.reshape(win * cin, 2, pout * cout)
        if pad_lanes > pout * cout:
            wd = jnp.pad(wd, ((0, 0), (0, 0), (0, pad_lanes - pout * cout)))
        blocks.append(wd.reshape(wd.shape[0], -1))
    return jnp.concatenate(blocks, axis=1)


def _fused_net_kernel(x_ref, w1_ref, b1_ref, w2_ref, b2_ref, w3_ref, b3_ref,
                      wf1_ref, bf1_ref, wf2_ref, bf2_ref, wf3_ref, bf3_ref,
                      o_ref):
    tb = x_ref.shape[0]

    x = x_ref[...].reshape(tb * 32, 96)
    z = jnp.dot(x, w1_ref[...], preferred_element_type=_F32)
    z = z.reshape(tb, 32, 768)
    y = z[:, 0:30, 0:256] + z[:, 1:31, 256:512] + z[:, 2:32, 512:768]
    y = jnp.max(y.reshape(tb, 15, 2, 256), axis=2)
    y = jnp.maximum(y[:, :, 0:128], y[:, :, 128:256])
    y = jnp.maximum(y + b1_ref[0], 0.0)

    zc = jnp.zeros((tb, 15, 16), _F32)
    x2 = jnp.concatenate([zc, y, zc[:, :, 0:8]], axis=2)
    zr = jnp.zeros((tb, 2, 152), _F32)
    x2 = jnp.concatenate([zr, x2, zr], axis=1)
    z = jnp.dot(x2.reshape(tb * 19, 152), w2_ref[...],
                preferred_element_type=_F32).reshape(tb, 19, 768)
    y = z[:, 0:17, 0:256] + z[:, 1:18, 256:512] + z[:, 2:19, 512:768]
    y = jnp.max(y[:, 0:16].reshape(tb, 8, 2, 256), axis=2)
    y = jnp.maximum(y[:, :, 0:128], y[:, :, 128:256])
    y = jnp.maximum(y + b2_ref[0], 0.0)

    zc = jnp.zeros((tb, 8, 32), _F32)
    x3 = jnp.concatenate([zc, y, zc], axis=2)
    zr = jnp.zeros((tb, 2, 192), _F32)
    x3 = jnp.concatenate([zr, x3, zr], axis=1)
    z = jnp.dot(x3.reshape(tb * 12, 192), w3_ref[...],
                preferred_element_type=_F32).reshape(tb, 12, 960)
    y = z[:, 0:10, 0:320] + z[:, 1:11, 320:640] + z[:, 2:12, 640:960]
    y = jnp.max(y.reshape(tb, 5, 2, 320), axis=2)
    y = jnp.maximum(y[:, :, 0:160], y[:, :, 160:320])
    y = jnp.maximum(y + b3_ref[0], 0.0)

    xf = jnp.concatenate([y[:, i, :] for i in range(5)], axis=1)
    h = jnp.dot(xf, wf1_ref[...], preferred_element_type=_F32)
    h = jnp.maximum(h + bf1_ref[0], 0.0)
    h = jnp.dot(h, wf2_ref[...], preferred_element_type=_F32)
    h = jnp.maximum(h + bf2_ref[0], 0.0)
    lg = jnp.dot(h, wf3_ref[...], preferred_element_type=_F32) + bf3_ref[0]
    m = jnp.max(lg, axis=1, keepdims=True)
    lse = m + jnp.log(jnp.sum(jnp.exp(lg - m), axis=1, keepdims=True))
    o_ref[...] = lg - lse


def kernel(x_nchw, w_c1, b_c1, w_c2, b_c2, w_c3, b_c3,
           w_fc1, b_fc1, w_fc2, b_fc2, w_fc3, b_fc3):
    B = x_nchw.shape[0]

    x1 = jnp.pad(x_nchw, ((0, 0), (0, 0), (2, 2), (2, 2)))
    x1 = x1.transpose(0, 2, 1, 3).reshape(B, 32, 96)

    w1 = _conv_toeplitz(w_c1, 3, 32, 15, cin_major=True, pad_lanes=128)
    w2 = _conv_toeplitz(w_c2, 8, 19, 8, cin_major=False, pad_lanes=128)
    w3 = _conv_toeplitz(w_c3, 16, 12, 5, cin_major=False, pad_lanes=160)

    b1 = jnp.pad(jnp.tile(b_c1.reshape(-1), 15), (0, 8)).reshape(1, 128)
    b2 = jnp.tile(b_c2.reshape(-1), 8).reshape(1, 128)
    b3 = jnp.tile(b_c3.reshape(-1), 5).reshape(1, 160)

    perm = np.array([co * 25 + py * 5 + px
                     for py in range(5) for px in range(5)
                     for co in range(32)], np.int32)
    wf1 = jnp.pad(w_fc1[perm], ((0, 0), (0, 24)))
    bf1 = jnp.pad(b_fc1, ((0, 0), (0, 24)))
    wf2 = jnp.pad(w_fc2, ((0, 24), (0, 0)))

    TB = 128
    G = B // TB
    nout = w_fc3.shape[1]

    out = pl.pallas_call(
        _fused_net_kernel,
        out_shape=jax.ShapeDtypeStruct((B, nout), _F32),
        grid=(G,),
        in_specs=[
            pl.BlockSpec((TB, 32, 96), lambda i: (i, 0, 0)),
            pl.BlockSpec((96, 768), lambda i: (0, 0)),
            pl.BlockSpec((1, 128), lambda i: (0, 0)),
            pl.BlockSpec((152, 768), lambda i: (0, 0)),
            pl.BlockSpec((1, 128), lambda i: (0, 0)),
            pl.BlockSpec((192, 960), lambda i: (0, 0)),
            pl.BlockSpec((1, 160), lambda i: (0, 0)),
            pl.BlockSpec((800, 1024), lambda i: (0, 0)),
            pl.BlockSpec((1, 1024), lambda i: (0, 0)),
            pl.BlockSpec((1024, 64), lambda i: (0, 0)),
            pl.BlockSpec((1, 64), lambda i: (0, 0)),
            pl.BlockSpec((64, nout), lambda i: (0, 0)),
            pl.BlockSpec((1, nout), lambda i: (0, 0)),
        ],
        out_specs=pl.BlockSpec((TB, nout), lambda i: (i, 0)),
        compiler_params=pltpu.CompilerParams(
            dimension_semantics=("parallel",)),
    )(x1, w1, b1, w2, b2, w3, b3, wf1, bf1, wf2, b_fc2, w_fc3, b_fc3)
    return out


# speedup vs baseline: 45.0509x; 1.0110x over previous
import numpy as np
import jax
import jax.numpy as jnp
from jax.experimental import pallas as pl
from jax.experimental.pallas import tpu as pltpu

_F32 = jnp.float32


def _shift_matrix(win, pout, ncol_pad):
    e = np.zeros((3, win, 2, pout), np.float32)
    for dx in range(3):
        for q in range(2):
            for p in range(pout):
                e[dx, 2 * p + q + dx, q, p] = 1.0
    return e


def _conv_toeplitz(w_mat, cin, win, pout, cin_major, pad_lanes):
    cout = w_mat.shape[1]
    taps = w_mat.reshape(3, 3, cin, cout)
    e = jnp.asarray(_shift_matrix(win, pout, None))
    blocks = []
    for dy in range(3):
        t = taps[dy]
        if cin_major:
            wd = jnp.einsum("dxqp,dco->cxqpo", e, t)
            wd = wd.reshape(cin * win, 2, pout * cout)
        else:
            wd = jnp.einsum("dxqp,dco->xcqpo", e, t)
            wd = wd.reshape(win * cin, 2, pout * cout)
        if pad_lanes > pout * cout:
            wd = jnp.pad(wd, ((0, 0), (0, 0), (0, pad_lanes - pout * cout)))
        blocks.append(wd.reshape(wd.shape[0], -1))
    return jnp.concatenate(blocks, axis=1)


def _fused_net_kernel(x_ref, w1_ref, b1_ref, w2_ref, b2_ref, w3_ref, b3_ref,
                      wf1_ref, bf1_ref, wf2_ref, bf2_ref, wf3_ref, bf3_ref,
                      o_ref):
    tb = x_ref.shape[0]

    x = x_ref[...].reshape(tb * 32, 96)
    z = jnp.dot(x, w1_ref[...], preferred_element_type=_F32)
    z = z.reshape(tb, 32, 768)
    y = z[:, 0:30, 0:256] + z[:, 1:31, 256:512] + z[:, 2:32, 512:768]
    y = jnp.max(y.reshape(tb, 15, 2, 256), axis=2)
    y = jnp.maximum(y[:, :, 0:128], y[:, :, 128:256])
    y = jnp.maximum(y + b1_ref[0], 0.0).astype(jnp.bfloat16)

    zc = jnp.zeros((tb, 15, 16), jnp.bfloat16)
    x2 = jnp.concatenate([zc, y, zc[:, :, 0:8]], axis=2)
    zr = jnp.zeros((tb, 2, 152), jnp.bfloat16)
    x2 = jnp.concatenate([zr, x2, zr], axis=1)
    z = jnp.dot(x2.reshape(tb * 19, 152), w2_ref[...],
                preferred_element_type=_F32).reshape(tb, 19, 768)
    y = z[:, 0:17, 0:256] + z[:, 1:18, 256:512] + z[:, 2:19, 512:768]
    y = jnp.max(y[:, 0:16].reshape(tb, 8, 2, 256), axis=2)
    y = jnp.maximum(y[:, :, 0:128], y[:, :, 128:256])
    y = jnp.maximum(y + b2_ref[0], 0.0).astype(jnp.bfloat16)

    zc = jnp.zeros((tb, 8, 32), jnp.bfloat16)
    x3 = jnp.concatenate([zc, y, zc], axis=2)
    zr = jnp.zeros((tb, 2, 192), jnp.bfloat16)
    x3 = jnp.concatenate([zr, x3, zr], axis=1)
    z = jnp.dot(x3.reshape(tb * 12, 192), w3_ref[...],
                preferred_element_type=_F32).reshape(tb, 12, 960)
    y = z[:, 0:10, 0:320] + z[:, 1:11, 320:640] + z[:, 2:12, 640:960]
    y = jnp.max(y.reshape(tb, 5, 2, 320), axis=2)
    y = jnp.maximum(y[:, :, 0:160], y[:, :, 160:320])
    y = jnp.maximum(y + b3_ref[0], 0.0).astype(jnp.bfloat16)

    xf = jnp.concatenate([y[:, i, :] for i in range(5)], axis=1)
    h = jnp.dot(xf, wf1_ref[...], preferred_element_type=_F32)
    h = jnp.maximum(h + bf1_ref[0], 0.0).astype(jnp.bfloat16)
    h = jnp.dot(h, wf2_ref[...], preferred_element_type=_F32)
    h = jnp.maximum(h + bf2_ref[0], 0.0).astype(jnp.bfloat16)
    lg = jnp.dot(h, wf3_ref[...], preferred_element_type=_F32) + bf3_ref[0]
    m = jnp.max(lg, axis=1, keepdims=True)
    lse = m + jnp.log(jnp.sum(jnp.exp(lg - m), axis=1, keepdims=True))
    o_ref[...] = lg - lse


def kernel(x_nchw, w_c1, b_c1, w_c2, b_c2, w_c3, b_c3,
           w_fc1, b_fc1, w_fc2, b_fc2, w_fc3, b_fc3):
    B = x_nchw.shape[0]

    x1 = jnp.pad(x_nchw, ((0, 0), (0, 0), (2, 2), (2, 2)))
    x1 = x1.transpose(0, 2, 1, 3).reshape(B, 32, 96).astype(jnp.bfloat16)

    bf16 = jnp.bfloat16
    w1 = _conv_toeplitz(w_c1, 3, 32, 15, cin_major=True, pad_lanes=128).astype(bf16)
    w2 = _conv_toeplitz(w_c2, 8, 19, 8, cin_major=False, pad_lanes=128).astype(bf16)
    w3 = _conv_toeplitz(w_c3, 16, 12, 5, cin_major=False, pad_lanes=160).astype(bf16)

    b1 = jnp.pad(jnp.tile(b_c1.reshape(-1), 15), (0, 8)).reshape(1, 128)
    b2 = jnp.tile(b_c2.reshape(-1), 8).reshape(1, 128)
    b3 = jnp.tile(b_c3.reshape(-1), 5).reshape(1, 160)

    perm = np.array([co * 25 + py * 5 + px
                     for py in range(5) for px in range(5)
                     for co in range(32)], np.int32)
    wf1 = jnp.pad(w_fc1[perm], ((0, 0), (0, 24))).astype(bf16)
    bf1 = jnp.pad(b_fc1, ((0, 0), (0, 24)))
    wf2 = jnp.pad(w_fc2, ((0, 24), (0, 0))).astype(bf16)

    TB = 128
    G = B // TB
    nout = w_fc3.shape[1]

    out = pl.pallas_call(
        _fused_net_kernel,
        out_shape=jax.ShapeDtypeStruct((B, nout), _F32),
        grid=(G,),
        in_specs=[
            pl.BlockSpec((TB, 32, 96), lambda i: (i, 0, 0)),
            pl.BlockSpec((96, 768), lambda i: (0, 0)),
            pl.BlockSpec((1, 128), lambda i: (0, 0)),
            pl.BlockSpec((152, 768), lambda i: (0, 0)),
            pl.BlockSpec((1, 128), lambda i: (0, 0)),
            pl.BlockSpec((192, 960), lambda i: (0, 0)),
            pl.BlockSpec((1, 160), lambda i: (0, 0)),
            pl.BlockSpec((800, 1024), lambda i: (0, 0)),
            pl.BlockSpec((1, 1024), lambda i: (0, 0)),
            pl.BlockSpec((1024, 64), lambda i: (0, 0)),
            pl.BlockSpec((1, 64), lambda i: (0, 0)),
            pl.BlockSpec((64, nout), lambda i: (0, 0)),
            pl.BlockSpec((1, nout), lambda i: (0, 0)),
        ],
        out_specs=pl.BlockSpec((TB, nout), lambda i: (i, 0)),
        compiler_params=pltpu.CompilerParams(
            dimension_semantics=("parallel",)),
    )(x1, w1, b1, w2, b2, w3, b3, wf1, bf1, wf2, b_fc2,
      w_fc3.astype(bf16), b_fc3)
    return out
